# baseline (speedup 1.0000x reference)
import jax
import jax.numpy as jnp
from jax import lax
from jax.experimental import pallas as pl
from jax.experimental.pallas import tpu as pltpu
from jax.experimental.pallas import tpu_sc as plsc

N_NODES = 10000
N_EDGES = 320000
D = 128

NC = 2
NS = 16
CHUNK = 100
EDGES_PER_TILE = N_EDGES // (NC * NS)
NCHUNK = EDGES_PER_TILE // CHUNK
N_PAD = 10240
ROWS_PER_TILE = N_PAD // NS
NBUF = 2

DEG_CHUNK = 125
DEG_NCHUNK = EDGES_PER_TILE // DEG_CHUNK
DEG_W = 16

_MESH = plsc.VectorSubcoreMesh(core_axis_name="c", subcore_axis_name="s")
_SC_PARAMS = pltpu.CompilerParams(use_tc_tiling_on_sc=False)


def _deg_body(dstd_hbm, zeros_hbm, ones_hbm, out_hbm, idx_v, ones_v, deg_sh,
              ssem):
    c = lax.axis_index("c")
    s = lax.axis_index("s")
    pltpu.sync_copy(dstd_hbm.at[c, s], idx_v)
    pltpu.sync_copy(ones_hbm, ones_v)
    pltpu.sync_copy(zeros_hbm, deg_sh.at[pl.ds(s * ROWS_PER_TILE, ROWS_PER_TILE)])
    plsc.subcore_barrier()

    def fire(j, carry):
        pltpu.async_copy(ones_v, deg_sh.at[idx_v.at[j]], ssem, add=True)
        return carry

    lax.fori_loop(0, DEG_NCHUNK, fire, 0)

    def drain(j, carry):
        pltpu.make_async_copy(ones_v, deg_sh.at[idx_v.at[0]], ssem).wait()
        return carry

    lax.fori_loop(0, DEG_NCHUNK, drain, 0)
    plsc.subcore_barrier()
    sl = pl.ds(s * ROWS_PER_TILE, ROWS_PER_TILE)
    pltpu.sync_copy(deg_sh.at[sl], out_hbm.at[c, sl])


def _deg_call(dstd_r, zeros_deg, ones_deg):
    return pl.kernel(
        _deg_body,
        out_type=jax.ShapeDtypeStruct((NC, N_PAD, DEG_W), jnp.float32),
        mesh=_MESH,
        compiler_params=_SC_PARAMS,
        scratch_types=[
            pltpu.VMEM((DEG_NCHUNK, DEG_CHUNK), jnp.int32),
            pltpu.VMEM((DEG_CHUNK, DEG_W), jnp.float32),
            pltpu.VMEM_SHARED((N_PAD, DEG_W), jnp.float32),
            pltpu.SemaphoreType.DMA,
        ],
    )(dstd_r, zeros_deg, ones_deg)


def _scatter_body(u_hbm, ei_hbm, zeros_hbm, out_hbm,
                  idx_v, rows, acc_sh, gsem):
    c = lax.axis_index("c")
    s = lax.axis_index("s")
    pltpu.sync_copy(ei_hbm.at[0, c, s], idx_v.at[0])
    pltpu.sync_copy(ei_hbm.at[1, c, s], idx_v.at[1])
    pltpu.sync_copy(zeros_hbm, acc_sh.at[pl.ds(s * ROWS_PER_TILE, ROWS_PER_TILE)])
    plsc.subcore_barrier()

    def body(j, carry):
        b = j % NBUF

        @pl.when(j >= NBUF)
        def _drain():
            pltpu.make_async_copy(
                u_hbm.at[idx_v.at[0, j - NBUF]], rows.at[b], gsem).wait()
            pltpu.sync_copy(rows.at[b], acc_sh.at[idx_v.at[1, j - NBUF]],
                            add=True)

        @pl.when(j < NCHUNK)
        def _launch():
            pltpu.async_copy(u_hbm.at[idx_v.at[0, j]], rows.at[b], gsem)

        return carry

    lax.fori_loop(0, NCHUNK + NBUF, body, 0)
    plsc.subcore_barrier()
    sl = pl.ds(s * ROWS_PER_TILE, ROWS_PER_TILE)
    pltpu.sync_copy(acc_sh.at[sl], out_hbm.at[c, sl])


def _scatter_call(u, ei_r, zeros_acc):
    return pl.kernel(
        _scatter_body,
        out_type=jax.ShapeDtypeStruct((NC, N_PAD, D), jnp.float32),
        mesh=_MESH,
        compiler_params=_SC_PARAMS,
        scratch_types=[
            pltpu.VMEM((2, NCHUNK, CHUNK), jnp.int32),
            pltpu.VMEM((NBUF, CHUNK, D), jnp.float32),
            pltpu.VMEM_SHARED((N_PAD, D), jnp.float32),
            pltpu.SemaphoreType.DMA,
        ],
    )(u, ei_r, zeros_acc)


_ROWS = 1000


def _mm1_body(x_ref, w_ref, deg_ref, u_ref, dis_ref):
    d = deg_ref[0, :, 0:1] + deg_ref[1, :, 0:1]
    dis = lax.rsqrt(1.0 + d)
    dis_ref[...] = dis
    u_ref[...] = jnp.dot(x_ref[...], w_ref[...],
                         preferred_element_type=jnp.float32) * dis


def _mm1_call(x, W1, degp):
    grid = (N_NODES // _ROWS,)
    return pl.pallas_call(
        _mm1_body,
        grid=grid,
        in_specs=[
            pl.BlockSpec((_ROWS, D), lambda i: (i, 0)),
            pl.BlockSpec((D, D), lambda i: (0, 0)),
            pl.BlockSpec((NC, _ROWS, DEG_W), lambda i: (0, i, 0)),
        ],
        out_specs=[
            pl.BlockSpec((_ROWS, D), lambda i: (i, 0)),
            pl.BlockSpec((_ROWS, 1), lambda i: (i, 0)),
        ],
        out_shape=[
            jax.ShapeDtypeStruct((N_NODES, D), jnp.float32),
            jax.ShapeDtypeStruct((N_NODES, 1), jnp.float32),
        ],
    )(x, W1, degp)


def _mid_body(s_ref, u_ref, dis_ref, b_ref, w_ref, o_ref):
    dis = dis_ref[...]
    z = jnp.maximum(
        dis * (s_ref[0] + s_ref[1] + u_ref[...]) + b_ref[...], 0.0)
    o_ref[...] = jnp.dot(z, w_ref[...],
                         preferred_element_type=jnp.float32) * dis


def _mid_call(sp, u, dis, b, Wnext):
    grid = (N_NODES // _ROWS,)
    return pl.pallas_call(
        _mid_body,
        grid=grid,
        in_specs=[
            pl.BlockSpec((NC, _ROWS, D), lambda i: (0, i, 0)),
            pl.BlockSpec((_ROWS, D), lambda i: (i, 0)),
            pl.BlockSpec((_ROWS, 1), lambda i: (i, 0)),
            pl.BlockSpec((1, D), lambda i: (0, 0)),
            pl.BlockSpec((D, D), lambda i: (0, 0)),
        ],
        out_specs=pl.BlockSpec((_ROWS, D), lambda i: (i, 0)),
        out_shape=jax.ShapeDtypeStruct((N_NODES, D), jnp.float32),
    )(sp, u, dis, b, Wnext)


def _final_body(s_ref, u_ref, dis_ref, b_ref, o_ref):
    o_ref[...] = dis_ref[...] * (s_ref[0] + s_ref[1] + u_ref[...]) \
        + b_ref[...]


def _final_call(sp, u, dis, b):
    grid = (N_NODES // _ROWS,)
    return pl.pallas_call(
        _final_body,
        grid=grid,
        in_specs=[
            pl.BlockSpec((NC, _ROWS, D), lambda i: (0, i, 0)),
            pl.BlockSpec((_ROWS, D), lambda i: (i, 0)),
            pl.BlockSpec((_ROWS, 1), lambda i: (i, 0)),
            pl.BlockSpec((1, D), lambda i: (0, 0)),
        ],
        out_specs=pl.BlockSpec((_ROWS, D), lambda i: (i, 0)),
        out_shape=jax.ShapeDtypeStruct((N_NODES, D), jnp.float32),
    )(sp, u, dis, b)


def kernel(x, edge_index, W1, b1, W2, b2, W3, b3):
    ei32 = edge_index.astype(jnp.int32)
    ei_r = ei32.reshape(2, NC, NS, NCHUNK, CHUNK)
    dstd_r = ei32[1].reshape(NC, NS, DEG_NCHUNK, DEG_CHUNK)

    zeros_acc = jnp.zeros((ROWS_PER_TILE, D), jnp.float32)
    zeros_deg = jnp.zeros((ROWS_PER_TILE, DEG_W), jnp.float32)
    ones_deg = jnp.ones((DEG_CHUNK, DEG_W), jnp.float32)

    degp = _deg_call(dstd_r, zeros_deg, ones_deg)

    b1r = b1.reshape(1, D)
    b2r = b2.reshape(1, D)
    b3r = b3.reshape(1, D)

    u1, dis = _mm1_call(x, W1, degp)
    s1 = _scatter_call(u1, ei_r, zeros_acc)
    u2 = _mid_call(s1, u1, dis, b1r, W2)
    s2 = _scatter_call(u2, ei_r, zeros_acc)
    u3 = _mid_call(s2, u2, dis, b2r, W3)
    s3 = _scatter_call(u3, ei_r, zeros_acc)
    return _final_call(s3, u3, dis, b3r)


# speedup vs baseline: 1.1408x; 1.1408x over previous
import jax
import jax.numpy as jnp
from jax import lax
from jax.experimental import pallas as pl
from jax.experimental.pallas import tpu as pltpu
from jax.experimental.pallas import tpu_sc as plsc

N_NODES = 10000
N_EDGES = 320000
D = 128

NC = 2
NS = 16
CHUNK = 40
EDGES_PER_TILE = N_EDGES // (NC * NS)
NCHUNK = EDGES_PER_TILE // CHUNK
N_PAD = 10240
ROWS_PER_TILE = N_PAD // NS
NBUF = 5

DEG_CHUNK = 125
DEG_NCHUNK = EDGES_PER_TILE // DEG_CHUNK
DEG_W = 16

_MESH = plsc.VectorSubcoreMesh(core_axis_name="c", subcore_axis_name="s")
_SC_PARAMS = pltpu.CompilerParams(use_tc_tiling_on_sc=False)


def _deg_body(dstd_hbm, zeros_hbm, ones_hbm, out_hbm, idx_v, ones_v, deg_sh,
              ssem):
    c = lax.axis_index("c")
    s = lax.axis_index("s")
    pltpu.sync_copy(dstd_hbm.at[c, s], idx_v)
    pltpu.sync_copy(ones_hbm, ones_v)
    pltpu.sync_copy(zeros_hbm, deg_sh.at[pl.ds(s * ROWS_PER_TILE, ROWS_PER_TILE)])
    plsc.subcore_barrier()

    def fire(j, carry):
        pltpu.async_copy(ones_v, deg_sh.at[idx_v.at[j]], ssem, add=True)
        return carry

    lax.fori_loop(0, DEG_NCHUNK, fire, 0)

    def drain(j, carry):
        pltpu.make_async_copy(ones_v, deg_sh.at[idx_v.at[0]], ssem).wait()
        return carry

    lax.fori_loop(0, DEG_NCHUNK, drain, 0)
    plsc.subcore_barrier()
    sl = pl.ds(s * ROWS_PER_TILE, ROWS_PER_TILE)
    pltpu.sync_copy(deg_sh.at[sl], out_hbm.at[c, sl])


def _deg_call(dstd_r, zeros_deg, ones_deg):
    return pl.kernel(
        _deg_body,
        out_type=jax.ShapeDtypeStruct((NC, N_PAD, DEG_W), jnp.float32),
        mesh=_MESH,
        compiler_params=_SC_PARAMS,
        scratch_types=[
            pltpu.VMEM((DEG_NCHUNK, DEG_CHUNK), jnp.int32),
            pltpu.VMEM((DEG_CHUNK, DEG_W), jnp.float32),
            pltpu.VMEM_SHARED((N_PAD, DEG_W), jnp.float32),
            pltpu.SemaphoreType.DMA,
        ],
    )(dstd_r, zeros_deg, ones_deg)


def _scatter_body(u_hbm, ei_hbm, zeros_hbm, out_hbm,
                  idx_v, rows, acc_sh, gsem):
    c = lax.axis_index("c")
    s = lax.axis_index("s")
    pltpu.sync_copy(ei_hbm.at[0, c, s], idx_v.at[0])
    pltpu.sync_copy(ei_hbm.at[1, c, s], idx_v.at[1])
    pltpu.sync_copy(zeros_hbm, acc_sh.at[pl.ds(s * ROWS_PER_TILE, ROWS_PER_TILE)])
    plsc.subcore_barrier()

    def body(j, carry):
        b = j % NBUF

        @pl.when(j >= NBUF)
        def _drain():
            pltpu.make_async_copy(
                u_hbm.at[idx_v.at[0, j - NBUF]], rows.at[b], gsem).wait()
            pltpu.sync_copy(rows.at[b], acc_sh.at[idx_v.at[1, j - NBUF]],
                            add=True)

        @pl.when(j < NCHUNK)
        def _launch():
            pltpu.async_copy(u_hbm.at[idx_v.at[0, j]], rows.at[b], gsem)

        return carry

    lax.fori_loop(0, NCHUNK + NBUF, body, 0)
    plsc.subcore_barrier()
    sl = pl.ds(s * ROWS_PER_TILE, ROWS_PER_TILE)
    pltpu.sync_copy(acc_sh.at[sl], out_hbm.at[c, sl])


def _scatter_call(u, ei_r, zeros_acc):
    return pl.kernel(
        _scatter_body,
        out_type=jax.ShapeDtypeStruct((NC, N_PAD, D), jnp.float32),
        mesh=_MESH,
        compiler_params=_SC_PARAMS,
        scratch_types=[
            pltpu.VMEM((2, NCHUNK, CHUNK), jnp.int32),
            pltpu.VMEM((NBUF, CHUNK, D), jnp.float32),
            pltpu.VMEM_SHARED((N_PAD, D), jnp.float32),
            pltpu.SemaphoreType.DMA,
        ],
    )(u, ei_r, zeros_acc)


_ROWS = 1000


def _mm1_body(x_ref, w_ref, deg_ref, u_ref, dis_ref):
    d = deg_ref[0, :, 0:1] + deg_ref[1, :, 0:1]
    dis = lax.rsqrt(1.0 + d)
    dis_ref[...] = dis
    u_ref[...] = jnp.dot(x_ref[...], w_ref[...],
                         preferred_element_type=jnp.float32) * dis


def _mm1_call(x, W1, degp):
    grid = (N_NODES // _ROWS,)
    return pl.pallas_call(
        _mm1_body,
        grid=grid,
        in_specs=[
            pl.BlockSpec((_ROWS, D), lambda i: (i, 0)),
            pl.BlockSpec((D, D), lambda i: (0, 0)),
            pl.BlockSpec((NC, _ROWS, DEG_W), lambda i: (0, i, 0)),
        ],
        out_specs=[
            pl.BlockSpec((_ROWS, D), lambda i: (i, 0)),
            pl.BlockSpec((_ROWS, 1), lambda i: (i, 0)),
        ],
        out_shape=[
            jax.ShapeDtypeStruct((N_NODES, D), jnp.float32),
            jax.ShapeDtypeStruct((N_NODES, 1), jnp.float32),
        ],
    )(x, W1, degp)


def _mid_body(s_ref, u_ref, dis_ref, b_ref, w_ref, o_ref):
    dis = dis_ref[...]
    z = jnp.maximum(
        dis * (s_ref[0] + s_ref[1] + u_ref[...]) + b_ref[...], 0.0)
    o_ref[...] = jnp.dot(z, w_ref[...],
                         preferred_element_type=jnp.float32) * dis


def _mid_call(sp, u, dis, b, Wnext):
    grid = (N_NODES // _ROWS,)
    return pl.pallas_call(
        _mid_body,
        grid=grid,
        in_specs=[
            pl.BlockSpec((NC, _ROWS, D), lambda i: (0, i, 0)),
            pl.BlockSpec((_ROWS, D), lambda i: (i, 0)),
            pl.BlockSpec((_ROWS, 1), lambda i: (i, 0)),
            pl.BlockSpec((1, D), lambda i: (0, 0)),
            pl.BlockSpec((D, D), lambda i: (0, 0)),
        ],
        out_specs=pl.BlockSpec((_ROWS, D), lambda i: (i, 0)),
        out_shape=jax.ShapeDtypeStruct((N_NODES, D), jnp.float32),
    )(sp, u, dis, b, Wnext)


def _final_body(s_ref, u_ref, dis_ref, b_ref, o_ref):
    o_ref[...] = dis_ref[...] * (s_ref[0] + s_ref[1] + u_ref[...]) \
        + b_ref[...]


def _final_call(sp, u, dis, b):
    grid = (N_NODES // _ROWS,)
    return pl.pallas_call(
        _final_body,
        grid=grid,
        in_specs=[
            pl.BlockSpec((NC, _ROWS, D), lambda i: (0, i, 0)),
            pl.BlockSpec((_ROWS, D), lambda i: (i, 0)),
            pl.BlockSpec((_ROWS, 1), lambda i: (i, 0)),
            pl.BlockSpec((1, D), lambda i: (0, 0)),
        ],
        out_specs=pl.BlockSpec((_ROWS, D), lambda i: (i, 0)),
        out_shape=jax.ShapeDtypeStruct((N_NODES, D), jnp.float32),
    )(sp, u, dis, b)


def kernel(x, edge_index, W1, b1, W2, b2, W3, b3):
    ei32 = edge_index.astype(jnp.int32)
    ei_r = ei32.reshape(2, NC, NS, NCHUNK, CHUNK)
    dstd_r = ei32[1].reshape(NC, NS, DEG_NCHUNK, DEG_CHUNK)

    zeros_acc = jnp.zeros((ROWS_PER_TILE, D), jnp.float32)
    zeros_deg = jnp.zeros((ROWS_PER_TILE, DEG_W), jnp.float32)
    ones_deg = jnp.ones((DEG_CHUNK, DEG_W), jnp.float32)

    degp = _deg_call(dstd_r, zeros_deg, ones_deg)

    b1r = b1.reshape(1, D)
    b2r = b2.reshape(1, D)
    b3r = b3.reshape(1, D)

    u1, dis = _mm1_call(x, W1, degp)
    s1 = _scatter_call(u1, ei_r, zeros_acc)
    u2 = _mid_call(s1, u1, dis, b1r, W2)
    s2 = _scatter_call(u2, ei_r, zeros_acc)
    u3 = _mid_call(s2, u2, dis, b2r, W3)
    s3 = _scatter_call(u3, ei_r, zeros_acc)
    return _final_call(s3, u3, dis, b3r)


# speedup vs baseline: 1.1464x; 1.0049x over previous
import jax
import jax.numpy as jnp
from jax import lax
from jax.experimental import pallas as pl
from jax.experimental.pallas import tpu as pltpu
from jax.experimental.pallas import tpu_sc as plsc

N_NODES = 10000
N_EDGES = 320000
D = 128

NC = 2
NS = 16
CHUNK = 40
EDGES_PER_TILE = N_EDGES // (NC * NS)
NCHUNK = EDGES_PER_TILE // CHUNK
N_PAD = 10000
ROWS_PER_TILE = N_PAD // NS
NBUF = 6

DEG_CHUNK = 125
DEG_NCHUNK = EDGES_PER_TILE // DEG_CHUNK
DEG_W = 16

_MESH = plsc.VectorSubcoreMesh(core_axis_name="c", subcore_axis_name="s")
_SC_PARAMS = pltpu.CompilerParams(use_tc_tiling_on_sc=False)


def _deg_body(dstd_hbm, zeros_hbm, ones_hbm, out_hbm, idx_v, ones_v, deg_sh,
              ssem):
    c = lax.axis_index("c")
    s = lax.axis_index("s")
    pltpu.sync_copy(dstd_hbm.at[c, s], idx_v)
    pltpu.sync_copy(ones_hbm, ones_v)
    pltpu.sync_copy(zeros_hbm, deg_sh.at[pl.ds(s * ROWS_PER_TILE, ROWS_PER_TILE)])
    plsc.subcore_barrier()

    def fire(j, carry):
        pltpu.async_copy(ones_v, deg_sh.at[idx_v.at[j]], ssem, add=True)
        return carry

    lax.fori_loop(0, DEG_NCHUNK, fire, 0)

    def drain(j, carry):
        pltpu.make_async_copy(ones_v, deg_sh.at[idx_v.at[0]], ssem).wait()
        return carry

    lax.fori_loop(0, DEG_NCHUNK, drain, 0)
    plsc.subcore_barrier()
    sl = pl.ds(s * ROWS_PER_TILE, ROWS_PER_TILE)
    pltpu.sync_copy(deg_sh.at[sl], out_hbm.at[c, sl])


def _deg_call(dstd_r, zeros_deg, ones_deg):
    return pl.kernel(
        _deg_body,
        out_type=jax.ShapeDtypeStruct((NC, N_PAD, DEG_W), jnp.float32),
        mesh=_MESH,
        compiler_params=_SC_PARAMS,
        scratch_types=[
            pltpu.VMEM((DEG_NCHUNK, DEG_CHUNK), jnp.int32),
            pltpu.VMEM((DEG_CHUNK, DEG_W), jnp.float32),
            pltpu.VMEM_SHARED((N_PAD, DEG_W), jnp.float32),
            pltpu.SemaphoreType.DMA,
        ],
    )(dstd_r, zeros_deg, ones_deg)


def _scatter_body(u_hbm, ei_hbm, zeros_hbm, out_hbm,
                  idx_v, rows, acc_sh, gsem):
    c = lax.axis_index("c")
    s = lax.axis_index("s")
    pltpu.sync_copy(ei_hbm.at[0, c, s], idx_v.at[0])
    pltpu.sync_copy(ei_hbm.at[1, c, s], idx_v.at[1])
    pltpu.sync_copy(zeros_hbm, acc_sh.at[pl.ds(s * ROWS_PER_TILE, ROWS_PER_TILE)])
    plsc.subcore_barrier()

    def body(j, carry):
        b = j % NBUF

        @pl.when(j >= NBUF)
        def _drain():
            pltpu.make_async_copy(
                u_hbm.at[idx_v.at[0, j - NBUF]], rows.at[b], gsem).wait()
            pltpu.sync_copy(rows.at[b], acc_sh.at[idx_v.at[1, j - NBUF]],
                            add=True)

        @pl.when(j < NCHUNK)
        def _launch():
            pltpu.async_copy(u_hbm.at[idx_v.at[0, j]], rows.at[b], gsem)

        return carry

    lax.fori_loop(0, NCHUNK + NBUF, body, 0)
    plsc.subcore_barrier()
    sl = pl.ds(s * ROWS_PER_TILE, ROWS_PER_TILE)
    pltpu.sync_copy(acc_sh.at[sl], out_hbm.at[c, sl])


def _scatter_call(u, ei_r, zeros_acc):
    return pl.kernel(
        _scatter_body,
        out_type=jax.ShapeDtypeStruct((NC, N_PAD, D), jnp.float32),
        mesh=_MESH,
        compiler_params=_SC_PARAMS,
        scratch_types=[
            pltpu.VMEM((2, NCHUNK, CHUNK), jnp.int32),
            pltpu.VMEM((NBUF, CHUNK, D), jnp.float32),
            pltpu.VMEM_SHARED((N_PAD, D), jnp.float32),
            pltpu.SemaphoreType.DMA,
        ],
    )(u, ei_r, zeros_acc)


_ROWS = 1000


def _mm1_body(x_ref, w_ref, deg_ref, u_ref, dis_ref):
    d = deg_ref[0, :, 0:1] + deg_ref[1, :, 0:1]
    dis = lax.rsqrt(1.0 + d)
    dis_ref[...] = dis
    u_ref[...] = jnp.dot(x_ref[...], w_ref[...],
                         preferred_element_type=jnp.float32) * dis


def _mm1_call(x, W1, degp):
    grid = (N_NODES // _ROWS,)
    return pl.pallas_call(
        _mm1_body,
        grid=grid,
        in_specs=[
            pl.BlockSpec((_ROWS, D), lambda i: (i, 0)),
            pl.BlockSpec((D, D), lambda i: (0, 0)),
            pl.BlockSpec((NC, _ROWS, DEG_W), lambda i: (0, i, 0)),
        ],
        out_specs=[
            pl.BlockSpec((_ROWS, D), lambda i: (i, 0)),
            pl.BlockSpec((_ROWS, 1), lambda i: (i, 0)),
        ],
        out_shape=[
            jax.ShapeDtypeStruct((N_NODES, D), jnp.float32),
            jax.ShapeDtypeStruct((N_NODES, 1), jnp.float32),
        ],
    )(x, W1, degp)


def _mid_body(s_ref, u_ref, dis_ref, b_ref, w_ref, o_ref):
    dis = dis_ref[...]
    z = jnp.maximum(
        dis * (s_ref[0] + s_ref[1] + u_ref[...]) + b_ref[...], 0.0)
    o_ref[...] = jnp.dot(z, w_ref[...],
                         preferred_element_type=jnp.float32) * dis


def _mid_call(sp, u, dis, b, Wnext):
    grid = (N_NODES // _ROWS,)
    return pl.pallas_call(
        _mid_body,
        grid=grid,
        in_specs=[
            pl.BlockSpec((NC, _ROWS, D), lambda i: (0, i, 0)),
            pl.BlockSpec((_ROWS, D), lambda i: (i, 0)),
            pl.BlockSpec((_ROWS, 1), lambda i: (i, 0)),
            pl.BlockSpec((1, D), lambda i: (0, 0)),
            pl.BlockSpec((D, D), lambda i: (0, 0)),
        ],
        out_specs=pl.BlockSpec((_ROWS, D), lambda i: (i, 0)),
        out_shape=jax.ShapeDtypeStruct((N_NODES, D), jnp.float32),
    )(sp, u, dis, b, Wnext)


def _final_body(s_ref, u_ref, dis_ref, b_ref, o_ref):
    o_ref[...] = dis_ref[...] * (s_ref[0] + s_ref[1] + u_ref[...]) \
        + b_ref[...]


def _final_call(sp, u, dis, b):
    grid = (N_NODES // _ROWS,)
    return pl.pallas_call(
        _final_body,
        grid=grid,
        in_specs=[
            pl.BlockSpec((NC, _ROWS, D), lambda i: (0, i, 0)),
            pl.BlockSpec((_ROWS, D), lambda i: (i, 0)),
            pl.BlockSpec((_ROWS, 1), lambda i: (i, 0)),
            pl.BlockSpec((1, D), lambda i: (0, 0)),
        ],
        out_specs=pl.BlockSpec((_ROWS, D), lambda i: (i, 0)),
        out_shape=jax.ShapeDtypeStruct((N_NODES, D), jnp.float32),
    )(sp, u, dis, b)


def kernel(x, edge_index, W1, b1, W2, b2, W3, b3):
    ei32 = edge_index.astype(jnp.int32)
    ei_r = ei32.reshape(2, NC, NS, NCHUNK, CHUNK)
    dstd_r = ei32[1].reshape(NC, NS, DEG_NCHUNK, DEG_CHUNK)

    zeros_acc = jnp.zeros((ROWS_PER_TILE, D), jnp.float32)
    zeros_deg = jnp.zeros((ROWS_PER_TILE, DEG_W), jnp.float32)
    ones_deg = jnp.ones((DEG_CHUNK, DEG_W), jnp.float32)

    degp = _deg_call(dstd_r, zeros_deg, ones_deg)

    b1r = b1.reshape(1, D)
    b2r = b2.reshape(1, D)
    b3r = b3.reshape(1, D)

    u1, dis = _mm1_call(x, W1, degp)
    s1 = _scatter_call(u1, ei_r, zeros_acc)
    u2 = _mid_call(s1, u1, dis, b1r, W2)
    s2 = _scatter_call(u2, ei_r, zeros_acc)
    u3 = _mid_call(s2, u2, dis, b2r, W3)
    s3 = _scatter_call(u3, ei_r, zeros_acc)
    return _final_call(s3, u3, dis, b3r)
